# Initial kernel scaffold; baseline (speedup 1.0000x reference)
import jax
import jax.numpy as jnp
from jax.experimental import pallas as pl


def kernel(x, table):
    raise NotImplementedError("write your pallas kernel here")


# speedup vs baseline: 1.8426x; 1.8426x over previous
import functools

import jax
import jax.numpy as jnp
from jax import lax
from jax.experimental import pallas as pl
from jax.experimental.pallas import tpu as pltpu
from jax.experimental.pallas import tpu_sc as plsc

NC = 2
NS = 16
NW = NC * NS
IW = 128
NCH = 8


def _build(V, D, R):
  rows_per_w = R // NW
  n_steps = rows_per_w // NCH
  mesh = plsc.VectorSubcoreMesh(core_axis_name="c", subcore_axis_name="s")

  @functools.partial(
      pl.kernel,
      out_type=jax.ShapeDtypeStruct((R, IW, D), jnp.float32),
      mesh=mesh,
      compiler_params=pltpu.CompilerParams(use_tc_tiling_on_sc=False),
      scratch_types=[
          pltpu.VMEM((NCH, IW), jnp.int32),
          pltpu.VMEM((NCH, IW, D), jnp.float32),
          pltpu.SemaphoreType.DMA,
      ],
  )
  def gather_kernel(x_hbm, tab_hbm, out_hbm, idx_v, rows_v, sem):
    wid = lax.axis_index("s") * NC + lax.axis_index("c")
    base = wid * rows_per_w

    def step(i, carry):
      r0 = base + i * NCH
      pltpu.sync_copy(x_hbm.at[pl.ds(r0, NCH)], idx_v)
      copies = [
          pltpu.async_copy(tab_hbm.at[idx_v.at[j]], rows_v.at[j], sem)
          for j in range(NCH)
      ]
      for c in copies:
        c.wait()
      pltpu.sync_copy(rows_v, out_hbm.at[pl.ds(r0, NCH)])
      return carry

    lax.fori_loop(0, n_steps, step, 0)

  return gather_kernel


def kernel(x, table):
  B, H = x.shape
  V, D = table.shape
  n = B * H
  assert n % (NW * NCH * IW) == 0
  R = n // IW
  xf = x.reshape(R, IW).astype(jnp.int32)
  out = _build(V, D, R)(xf, table)
  return out.reshape(B, H, D)


# speedup vs baseline: 1.8759x; 1.0181x over previous
import functools

import jax
import jax.numpy as jnp
from jax import lax
from jax.experimental import pallas as pl
from jax.experimental.pallas import tpu as pltpu
from jax.experimental.pallas import tpu_sc as plsc

NC = 2
NS = 16
NW = NC * NS
IW = 128
NCH = 2
NBUF = 5


def _build(V, D, R):
  rows_per_w = R // NW
  n_steps = rows_per_w // NCH
  assert n_steps % NBUF == 0 and n_steps >= 2 * NBUF
  mesh = plsc.VectorSubcoreMesh(core_axis_name="c", subcore_axis_name="s")

  @functools.partial(
      pl.kernel,
      out_type=jax.ShapeDtypeStruct((R, IW, D), jnp.float32),
      mesh=mesh,
      compiler_params=pltpu.CompilerParams(use_tc_tiling_on_sc=False),
      scratch_types=[
          pltpu.VMEM((rows_per_w, IW), jnp.int32),
          pltpu.VMEM((NBUF, NCH, IW, D), jnp.float32),
          [pltpu.SemaphoreType.DMA] * NBUF,
          [pltpu.SemaphoreType.DMA] * NBUF,
      ],
  )
  def gather_kernel(x_hbm, tab_hbm, out_hbm, idx_v, rows_v, gsems, ssems):
    wid = lax.axis_index("s") * NC + lax.axis_index("c")
    base = wid * rows_per_w
    pltpu.sync_copy(x_hbm.at[pl.ds(base, rows_per_w)], idx_v)

    def issue_gathers(i, b):
      for j in range(NCH):
        pltpu.async_copy(
            tab_hbm.at[idx_v.at[i * NCH + j]], rows_v.at[b, j], gsems[b])

    def wait_gathers(i, b):
      for j in range(NCH):
        pltpu.make_async_copy(
            tab_hbm.at[idx_v.at[i * NCH + j]], rows_v.at[b, j],
            gsems[b]).wait()

    def store(i, b):
      return pltpu.make_async_copy(
          rows_v.at[b], out_hbm.at[pl.ds(base + i * NCH, NCH)], ssems[b])

    for k in range(NBUF - 1):
      issue_gathers(k, k)

    def group(g, carry):
      for k in range(NBUF):
        i = g * NBUF + k
        bb = (k - 1) % NBUF

        @pl.when(i > 0)
        def _():
          store(i - 1, bb).wait()

        @pl.when(i + NBUF - 1 < n_steps)
        def _():
          issue_gathers(i + NBUF - 1, bb)

        wait_gathers(i, k)
        store(i, k).start()
      return carry

    lax.fori_loop(0, n_steps // NBUF, group, 0)
    store(n_steps - 1, NBUF - 1).wait()

  return gather_kernel


def kernel(x, table):
  B, H = x.shape
  V, D = table.shape
  n = B * H
  assert n % (NW * NCH * IW) == 0
  R = n // IW
  xf = x.reshape(R, IW).astype(jnp.int32)
  out = _build(V, D, R)(xf, table)
  return out.reshape(B, H, D)
